# baseline (speedup 1.0000x reference)
import jax
import jax.numpy as jnp
from jax.experimental import pallas as pl
from jax.experimental.pallas import tpu as pltpu

_N_CH = 16


def _copy_body(pe_hbm, out_hbm, buf, rsem, wsem):
    rows = out_hbm.shape[0]
    ch = rows // _N_CH
    reads = []
    for c in range(_N_CH):
        cp = pltpu.make_async_copy(
            pe_hbm.at[pl.ds(c * ch, ch)],
            buf.at[pl.ds(c * ch, ch)],
            rsem,
        )
        cp.start()
        reads.append(cp)
    writes = []
    for c in range(_N_CH):
        reads[c].wait()
        wp = pltpu.make_async_copy(
            buf.at[pl.ds(c * ch, ch)],
            out_hbm.at[pl.ds(c * ch, ch)],
            wsem,
        )
        wp.start()
        writes.append(wp)
    for wp in writes:
        wp.wait()


def kernel(x, pe):
    seq_len = x.shape[1]
    d = pe.shape[2]
    pe2 = pe.reshape(pe.shape[1], d)
    out = pl.pallas_call(
        _copy_body,
        in_specs=[pl.BlockSpec(memory_space=pltpu.MemorySpace.HBM)],
        out_specs=pl.BlockSpec(memory_space=pltpu.MemorySpace.HBM),
        out_shape=jax.ShapeDtypeStruct((seq_len, d), pe.dtype),
        scratch_shapes=[
            pltpu.VMEM((seq_len, d), pe.dtype),
            pltpu.SemaphoreType.DMA,
            pltpu.SemaphoreType.DMA,
        ],
    )(pe2)
    return out.reshape(1, seq_len, d)
